# baseline (speedup 1.0000x reference)
import functools

import jax
import jax.numpy as jnp
from jax import lax
from jax.experimental import pallas as pl
from jax.experimental.pallas import tpu as pltpu
from jax.experimental.pallas import tpu_sc as plsc

EMBED = 768
SEQ = 2048
BATCH = 4
NW = 32
POSW = SEQ // NW
WIN = 8
NWIN = POSW // WIN
LANES = 16
EMB_VECS = EMBED // LANES

_mesh = plsc.VectorSubcoreMesh(core_axis_name="c", subcore_axis_name="s")


@functools.partial(
    pl.kernel,
    out_type=jax.ShapeDtypeStruct((BATCH, SEQ, EMBED), jnp.float32),
    mesh=_mesh,
    scratch_types=[
        pltpu.VMEM((NWIN, BATCH * WIN), jnp.int32),
        pltpu.VMEM((2, BATCH * WIN, EMBED), jnp.float32),
        pltpu.VMEM((2, BATCH * WIN, EMBED), jnp.float32),
        pltpu.VMEM((2, WIN, EMBED), jnp.float32),
        (pltpu.SemaphoreType.DMA,) * 2,
        (pltpu.SemaphoreType.DMA,) * 2,
        (pltpu.SemaphoreType.DMA,) * 2,
    ],
)
def _embed_add(ids_hbm, wte_hbm, wpe_hbm, out_hbm,
               idx_v, tok_v, stg_v, pos_v, sem_idx, sem_in, sem_out):
    wid = lax.axis_index("s") * 2 + lax.axis_index("c")
    p0 = wid * POSW

    idx_copies = [
        pltpu.async_copy(ids_hbm.at[b, pl.ds(p0 + w * WIN, WIN)],
                         idx_v.at[w, pl.ds(b * WIN, WIN)],
                         sem_idx[min(w, 1)])
        for w in range(NWIN) for b in range(BATCH)
    ]

    def in_descs(w, slot, make):
        mk = pltpu.make_async_copy if make else pltpu.async_copy
        return [
            mk(wte_hbm.at[idx_v.at[w]], tok_v.at[slot], sem_in[slot]),
            mk(wpe_hbm.at[pl.ds(p0 + w * WIN, WIN)], pos_v.at[slot],
               sem_in[slot]),
        ]

    def out_descs(w, slot, make):
        mk = pltpu.make_async_copy if make else pltpu.async_copy
        return [
            mk(stg_v.at[slot, pl.ds(b * WIN, WIN)],
               out_hbm.at[b, pl.ds(p0 + w * WIN, WIN), :], sem_out[slot])
            for b in range(BATCH)
        ]

    def wait_all(descs):
        for cp in descs:
            cp.wait()

    for cp in idx_copies[:BATCH]:
        cp.wait()
    in_descs(0, 0, make=False)
    for cp in idx_copies[BATCH:]:
        cp.wait()
    in_descs(1, 1, make=False)

    def add_loop(slot):
        def row_add(r):
            for k in range(EMB_VECS):
                sl = pl.ds(k * LANES, LANES)
                pv = pos_v[slot, r, sl]
                for b in range(BATCH):
                    stg_v[slot, b * WIN + r, sl] = (
                        tok_v[slot, b * WIN + r, sl] + pv)

        pl.loop(0, WIN, unroll=4)(row_add)

    def step(w, slot):
        wait_all(in_descs(w, slot, make=True))

        @pl.when(w >= 2)
        def _drain():
            wait_all(out_descs(w - 2, slot, make=True))

        add_loop(slot)

        @pl.when(w + 2 < NWIN)
        def _prefetch():
            in_descs(w + 2, slot, make=False)

        out_descs(w, slot, make=False)

    def pair(w0):
        step(w0, 0)
        step(w0 + 1, 1)

    pl.loop(0, NWIN, step=2, unroll=1)(pair)
    wait_all(out_descs(NWIN - 2, 0, make=True))
    wait_all(out_descs(NWIN - 1, 1, make=True))


def kernel(input_ids, wte, wpe):
    ids = input_ids.astype(jnp.int32)
    return _embed_add(ids, wte, wpe)


# speedup vs baseline: 1.7897x; 1.7897x over previous
import functools

import jax
import jax.numpy as jnp
from jax import lax
from jax.experimental import pallas as pl
from jax.experimental.pallas import tpu as pltpu
from jax.experimental.pallas import tpu_sc as plsc

EMBED = 768
SEQ = 2048
BATCH = 4
NW = 32
POSW = SEQ // NW
WIN = 8
NWIN = POSW // WIN
LANES = 16
EMB_VECS = EMBED // LANES

_mesh = plsc.VectorSubcoreMesh(core_axis_name="c", subcore_axis_name="s")


@functools.partial(
    pl.kernel,
    out_type=jax.ShapeDtypeStruct((BATCH, SEQ, EMBED), jnp.float32),
    mesh=_mesh,
    scratch_types=[
        pltpu.VMEM((NWIN, BATCH * WIN), jnp.int32),
        pltpu.VMEM((2, BATCH * WIN, EMBED), jnp.float32),
        pltpu.VMEM((2, BATCH * WIN, EMBED), jnp.float32),
        pltpu.VMEM((2, WIN, EMBED), jnp.float32),
        (pltpu.SemaphoreType.DMA,) * 2,
        (pltpu.SemaphoreType.DMA,) * 2,
        (pltpu.SemaphoreType.DMA,) * 2,
    ],
)
def _embed_add(ids_hbm, wte_hbm, wpe_hbm, out_hbm,
               idx_v, tok_v, stg_v, pos_v, sem_idx, sem_in, sem_out):
    wid = lax.axis_index("s") * 2 + lax.axis_index("c")
    p0 = wid * POSW

    idx_copies = [
        pltpu.async_copy(ids_hbm.at[b, pl.ds(p0 + w * WIN, WIN)],
                         idx_v.at[w, pl.ds(b * WIN, WIN)],
                         sem_idx[min(w, 1)])
        for w in range(NWIN) for b in range(BATCH)
    ]

    def in_descs(w, slot, make):
        mk = pltpu.make_async_copy if make else pltpu.async_copy
        return [
            mk(wte_hbm.at[idx_v.at[w]], tok_v.at[slot], sem_in[slot]),
            mk(wpe_hbm.at[pl.ds(p0 + w * WIN, WIN)], pos_v.at[slot],
               sem_in[slot]),
        ]

    def out_descs(w, slot, make):
        mk = pltpu.make_async_copy if make else pltpu.async_copy
        return [
            mk(stg_v.at[slot, pl.ds(b * WIN, WIN)],
               out_hbm.at[b, pl.ds(p0 + w * WIN, WIN), :], sem_out[slot])
            for b in range(BATCH)
        ]

    def wait_all(descs):
        for cp in descs:
            cp.wait()

    for cp in idx_copies[:BATCH]:
        cp.wait()
    in_descs(0, 0, make=False)
    for cp in idx_copies[BATCH:]:
        cp.wait()
    in_descs(1, 1, make=False)

    def add_loop(slot):
        def row_add(r):
            for k in range(EMB_VECS):
                sl = pl.ds(k * LANES, LANES)
                pv = pos_v[slot, r, sl]
                for b in range(BATCH):
                    stg_v[slot, b * WIN + r, sl] = (
                        tok_v[slot, b * WIN + r, sl] + pv)

        pl.loop(0, WIN, unroll=2)(row_add)

    def step(w, slot):
        wait_all(in_descs(w, slot, make=True))

        @pl.when(w >= 2)
        def _drain():
            wait_all(out_descs(w - 2, slot, make=True))

        add_loop(slot)

        @pl.when(w + 2 < NWIN)
        def _prefetch():
            in_descs(w + 2, slot, make=False)

        out_descs(w, slot, make=False)

    def pair(w0):
        step(w0, 0)
        step(w0 + 1, 1)

    pl.loop(0, NWIN, step=2, unroll=1)(pair)
    wait_all(out_descs(NWIN - 2, 0, make=True))
    wait_all(out_descs(NWIN - 1, 1, make=True))


def kernel(input_ids, wte, wpe):
    ids = input_ids.astype(jnp.int32)
    return _embed_add(ids, wte, wpe)


# speedup vs baseline: 1.7910x; 1.0007x over previous
import functools

import jax
import jax.numpy as jnp
from jax import lax
from jax.experimental import pallas as pl
from jax.experimental.pallas import tpu as pltpu
from jax.experimental.pallas import tpu_sc as plsc

EMBED = 768
SEQ = 2048
BATCH = 4
NW = 32
POSW = SEQ // NW
WIN = 8
NWIN = POSW // WIN
LANES = 16
EMB_VECS = EMBED // LANES

_mesh = plsc.VectorSubcoreMesh(core_axis_name="c", subcore_axis_name="s")


@functools.partial(
    pl.kernel,
    out_type=jax.ShapeDtypeStruct((BATCH, SEQ, EMBED), jnp.float32),
    mesh=_mesh,
    scratch_types=[
        pltpu.VMEM((NWIN, BATCH * WIN), jnp.int32),
        pltpu.VMEM((2, BATCH * WIN, EMBED), jnp.float32),
        pltpu.VMEM((2, BATCH, WIN, EMBED), jnp.float32),
        pltpu.VMEM((2, WIN, EMBED), jnp.float32),
        (pltpu.SemaphoreType.DMA,) * 2,
        (pltpu.SemaphoreType.DMA,) * 2,
        (pltpu.SemaphoreType.DMA,) * 2,
    ],
)
def _embed_add(ids_hbm, wte_hbm, wpe_hbm, out_hbm,
               idx_v, tok_v, stg_v, pos_v, sem_idx, sem_in, sem_out):
    wid = lax.axis_index("s") * 2 + lax.axis_index("c")
    p0 = wid * POSW

    idx_copies = [
        pltpu.async_copy(ids_hbm.at[b, pl.ds(p0 + w * WIN, WIN)],
                         idx_v.at[w, pl.ds(b * WIN, WIN)],
                         sem_idx[min(w, 1)])
        for w in range(NWIN) for b in range(BATCH)
    ]

    def in_descs(w, slot, make):
        mk = pltpu.make_async_copy if make else pltpu.async_copy
        return [
            mk(wte_hbm.at[idx_v.at[w]], tok_v.at[slot], sem_in[slot]),
            mk(wpe_hbm.at[pl.ds(p0 + w * WIN, WIN)], pos_v.at[slot],
               sem_in[slot]),
        ]

    def out_descs(w, slot, make):
        mk = pltpu.make_async_copy if make else pltpu.async_copy
        return [
            mk(stg_v.at[slot],
               out_hbm.at[:, pl.ds(p0 + w * WIN, WIN), :], sem_out[slot]),
        ]

    def wait_all(descs):
        for cp in descs:
            cp.wait()

    for cp in idx_copies[:BATCH]:
        cp.wait()
    in_descs(0, 0, make=False)
    for cp in idx_copies[BATCH:]:
        cp.wait()
    in_descs(1, 1, make=False)

    def add_loop(slot):
        def row_add(r):
            for k in range(EMB_VECS):
                sl = pl.ds(k * LANES, LANES)
                pv = pos_v[slot, r, sl]
                for b in range(BATCH):
                    stg_v[slot, b, r, sl] = (
                        tok_v[slot, b * WIN + r, sl] + pv)

        pl.loop(0, WIN, unroll=2)(row_add)

    def step(w, slot):
        wait_all(in_descs(w, slot, make=True))

        @pl.when(w >= 2)
        def _drain():
            wait_all(out_descs(w - 2, slot, make=True))

        add_loop(slot)

        @pl.when(w + 2 < NWIN)
        def _prefetch():
            in_descs(w + 2, slot, make=False)

        out_descs(w, slot, make=False)

    def pair(w0):
        step(w0, 0)
        step(w0 + 1, 1)

    pl.loop(0, NWIN, step=2, unroll=1)(pair)
    wait_all(out_descs(NWIN - 2, 0, make=True))
    wait_all(out_descs(NWIN - 1, 1, make=True))


def kernel(input_ids, wte, wpe):
    ids = input_ids.astype(jnp.int32)
    return _embed_add(ids, wte, wpe)


# speedup vs baseline: 1.7918x; 1.0005x over previous
import functools

import jax
import jax.numpy as jnp
from jax import lax
from jax.experimental import pallas as pl
from jax.experimental.pallas import tpu as pltpu
from jax.experimental.pallas import tpu_sc as plsc

EMBED = 768
SEQ = 2048
BATCH = 4
NW = 32
POSW = SEQ // NW
WIN = 8
NWIN = POSW // WIN
LANES = 16
EMB_VECS = EMBED // LANES

_mesh = plsc.VectorSubcoreMesh(core_axis_name="c", subcore_axis_name="s")


@functools.partial(
    pl.kernel,
    out_type=jax.ShapeDtypeStruct((BATCH, SEQ, EMBED), jnp.float32),
    mesh=_mesh,
    scratch_types=[
        pltpu.VMEM((NWIN, BATCH * WIN), jnp.int32),
        pltpu.VMEM((2, BATCH * WIN, EMBED), jnp.float32),
        pltpu.VMEM((2, BATCH * WIN, EMBED), jnp.float32),
        pltpu.VMEM((2, WIN, EMBED), jnp.float32),
        (pltpu.SemaphoreType.DMA,) * 2,
        (pltpu.SemaphoreType.DMA,) * 2,
        (pltpu.SemaphoreType.DMA,) * 2,
    ],
)
def _embed_add(ids_hbm, wte_hbm, wpe_hbm, out_hbm,
               idx_v, tok_v, stg_v, pos_v, sem_idx, sem_in, sem_out):
    wid = lax.axis_index("s") * 2 + lax.axis_index("c")
    p0 = wid * POSW

    idx_copies = [
        pltpu.async_copy(ids_hbm.at[b, pl.ds(p0 + w * WIN, WIN)],
                         idx_v.at[w, pl.ds(b * WIN, WIN)],
                         sem_idx[min(w, 1)])
        for w in range(NWIN) for b in range(BATCH)
    ]

    def in_descs(w, slot, make):
        mk = pltpu.make_async_copy if make else pltpu.async_copy
        return [
            mk(wte_hbm.at[idx_v.at[w]], tok_v.at[slot], sem_in[slot]),
            mk(wpe_hbm.at[pl.ds(p0 + w * WIN, WIN)], pos_v.at[slot],
               sem_in[slot]),
        ]

    def out_descs(w, slot, make):
        mk = pltpu.make_async_copy if make else pltpu.async_copy
        return [
            mk(stg_v.at[slot, pl.ds(b * WIN, WIN)],
               out_hbm.at[b, pl.ds(p0 + w * WIN, WIN), :], sem_out[slot])
            for b in range(BATCH)
        ]

    def wait_all(descs):
        for cp in descs:
            cp.wait()

    for cp in idx_copies[:BATCH]:
        cp.wait()
    in_descs(0, 0, make=False)
    for cp in idx_copies[BATCH:]:
        cp.wait()
    in_descs(1, 1, make=False)

    def add_loop(slot):
        def row_add(r):
            for k in range(EMB_VECS):
                sl = pl.ds(k * LANES, LANES)
                pv = pos_v[slot, r, sl]
                for b in range(BATCH):
                    stg_v[slot, b * WIN + r, sl] = (
                        tok_v[slot, b * WIN + r, sl] + pv)

        pl.loop(0, WIN, unroll=2)(row_add)

    def step(w, slot):
        wait_all(in_descs(w, slot, make=True))

        @pl.when(w >= 2)
        def _drain():
            wait_all(out_descs(w - 2, slot, make=True))

        add_loop(slot)

        @pl.when(w + 2 < NWIN)
        def _prefetch():
            in_descs(w + 2, slot, make=False)

        out_descs(w, slot, make=False)

    def pair(w0):
        step(w0, 0)
        step(w0 + 1, 1)

    pl.loop(0, NWIN, step=2, unroll=1)(pair)
    wait_all(out_descs(NWIN - 2, 0, make=True))
    wait_all(out_descs(NWIN - 1, 1, make=True))


def kernel(input_ids, wte, wpe):
    ids = input_ids.astype(jnp.int32)
    return _embed_add(ids, wte, wpe)
